# baseline (speedup 1.0000x reference)
import jax
import jax.numpy as jnp
from jax.experimental import pallas as pl
from jax.experimental.pallas import tpu as pltpu

_VMEM_LIMIT = 48 * 1024 * 1024


_CHUNK = 256


def _tile_compute(seg_tile, x_buf, tbl, o_buf, slot):
    tn = seg_tile.shape[0]
    s = tbl.shape[0]
    w = jnp.concatenate(
        [jnp.full((1, 128), 256.0, jnp.float32), jnp.ones((1, 128), jnp.float32)],
        axis=0)
    iota128 = jax.lax.broadcasted_iota(
        jnp.int32, (_CHUNK, 128), 1).astype(jnp.float32)
    for c in range(0, tn, _CHUNK):
        seg_b = jnp.dot(seg_tile[c:c + _CHUNK, :], w,
                        preferred_element_type=jnp.float32)
        groups = [(iota128 + float(g * 128) == seg_b).astype(jnp.float32)
                  for g in range(s // 128)]
        onehot = jnp.concatenate(groups, axis=1)
        emb = jnp.dot(onehot, tbl, preferred_element_type=jnp.float32)
        o_buf[slot, c:c + _CHUNK, :] = x_buf[slot, c:c + _CHUNK, :] + emb


def _pipeline_kernel(seg_hbm, x_hbm, tbl_hbm, o_hbm,
                     x_buf, o_buf, seg_vmem, tbl_vmem,
                     in_sem, out_sem, cst_sem, *, tn, n_steps):
    def dma_in(slot, step):
        pltpu.make_async_copy(x_hbm.at[pl.ds(step * tn, tn)],
                              x_buf.at[slot], in_sem.at[slot]).start()

    def wait_in(slot):
        pltpu.make_async_copy(x_hbm.at[pl.ds(0, tn)],
                              x_buf.at[slot], in_sem.at[slot]).wait()

    def dma_out(slot, step):
        pltpu.make_async_copy(o_buf.at[slot],
                              o_hbm.at[pl.ds(step * tn, tn)],
                              out_sem.at[slot]).start()

    def wait_out(slot):
        pltpu.make_async_copy(o_buf.at[slot],
                              o_hbm.at[pl.ds(0, tn)], out_sem.at[slot]).wait()

    pltpu.make_async_copy(seg_hbm, seg_vmem, cst_sem.at[0]).start()
    pltpu.make_async_copy(tbl_hbm, tbl_vmem, cst_sem.at[1]).start()
    dma_in(0, 0)
    pltpu.make_async_copy(seg_hbm, seg_vmem, cst_sem.at[0]).wait()
    pltpu.make_async_copy(tbl_hbm, tbl_vmem, cst_sem.at[1]).wait()
    tbl = tbl_vmem[...]

    def body(step, _):
        cur = jax.lax.rem(step, 2)
        nxt = jax.lax.rem(step + 1, 2)

        @pl.when(step + 1 < n_steps)
        def _():
            dma_in(nxt, step + 1)

        wait_in(cur)

        @pl.when(step >= 2)
        def _():
            wait_out(cur)

        base = pl.multiple_of(step * tn, tn)
        seg_tile = seg_vmem[pl.ds(base, tn), :]
        _tile_compute(seg_tile, x_buf, tbl, o_buf, cur)
        dma_out(cur, step)
        return ()

    jax.lax.fori_loop(0, n_steps, body, ())
    wait_out(jax.lax.rem(n_steps - 2, 2))
    wait_out(jax.lax.rem(n_steps - 1, 2))


def _pick_tile(n):
    for tn in (2048, 1024, 512, 256, 128, 64, 32, 16, 8):
        if n % tn == 0:
            return tn
    return n


def kernel(x, segment_ids, seg_embed):
    import functools

    L, B, D = x.shape
    N = L * B
    S = seg_embed.shape[0]
    tn = _pick_tile(N)
    n_steps = N // tn

    x2d = x.reshape(N, D)
    seg = segment_ids.reshape(N).astype(jnp.int32)
    seg2 = jnp.stack([(seg >> 8).astype(jnp.float32),
                      (seg & 255).astype(jnp.float32)], axis=-1)

    out2d = pl.pallas_call(
        functools.partial(_pipeline_kernel, tn=tn, n_steps=n_steps),
        out_shape=jax.ShapeDtypeStruct((N, D), x.dtype),
        in_specs=[
            pl.BlockSpec(memory_space=pl.ANY),
            pl.BlockSpec(memory_space=pl.ANY),
            pl.BlockSpec(memory_space=pl.ANY),
        ],
        out_specs=pl.BlockSpec(memory_space=pl.ANY),
        scratch_shapes=[
            pltpu.VMEM((2, tn, D), jnp.float32),
            pltpu.VMEM((2, tn, D), jnp.float32),
            pltpu.VMEM((N, 2), jnp.float32),
            pltpu.VMEM((S, D), jnp.float32),
            pltpu.SemaphoreType.DMA((2,)),
            pltpu.SemaphoreType.DMA((2,)),
            pltpu.SemaphoreType.DMA((2,)),
        ],
        compiler_params=pltpu.CompilerParams(
            vmem_limit_bytes=_VMEM_LIMIT),
    )(seg2, x2d, seg_embed)
    return out2d.reshape(L, B, D)


# speedup vs baseline: 1.1069x; 1.1069x over previous
import jax
import jax.numpy as jnp
from jax.experimental import pallas as pl
from jax.experimental.pallas import tpu as pltpu

_VMEM_LIMIT = 48 * 1024 * 1024


def _seg_add_kernel(seg_ref, x_ref, tbl_ref, o_ref):
    seg2 = seg_ref[...]
    tn = seg2.shape[0]
    s = tbl_ref.shape[0]
    w = jnp.concatenate(
        [jnp.full((1, 128), 256.0, jnp.float32), jnp.ones((1, 128), jnp.float32)],
        axis=0)
    seg_b = jnp.dot(seg2, w,
                    preferred_element_type=jnp.float32)
    iota128 = jax.lax.broadcasted_iota(jnp.int32, (tn, 128), 1).astype(jnp.float32)
    groups = [(iota128 + float(g * 128) == seg_b).astype(jnp.float32)
              for g in range(s // 128)]
    onehot = jnp.concatenate(groups, axis=1)
    emb = jnp.dot(onehot, tbl_ref[...],
                  preferred_element_type=jnp.float32)
    o_ref[...] = x_ref[...] + emb


def _pick_tile(n):
    for tn in (4096, 2048, 1024, 512, 256, 128, 64, 32, 16, 8):
        if n % tn == 0:
            return tn
    return n


def kernel(x, segment_ids, seg_embed):
    L, B, D = x.shape
    N = L * B
    S = seg_embed.shape[0]
    tn = _pick_tile(N)

    x2d = x.reshape(N, D)
    seg = segment_ids.reshape(N).astype(jnp.int32)
    seg2 = jnp.stack([(seg >> 8).astype(jnp.float32),
                      (seg & 255).astype(jnp.float32)], axis=-1)

    out2d = pl.pallas_call(
        _seg_add_kernel,
        out_shape=jax.ShapeDtypeStruct((N, D), x.dtype),
        grid=(N // tn,),
        in_specs=[
            pl.BlockSpec((tn, 2), lambda i: (i, 0)),
            pl.BlockSpec((tn, D), lambda i: (i, 0)),
            pl.BlockSpec((S, D), lambda i: (0, 0)),
        ],
        out_specs=pl.BlockSpec((tn, D), lambda i: (i, 0)),
        compiler_params=pltpu.CompilerParams(
            dimension_semantics=("parallel",),
            vmem_limit_bytes=_VMEM_LIMIT),
    )(seg2, x2d, seg_embed)
    return out2d.reshape(L, B, D)
